# Initial kernel scaffold; baseline (speedup 1.0000x reference)
import functools

import jax
import jax.numpy as jnp
from jax import lax
from jax.experimental import pallas as pl
from jax.experimental.pallas import tpu as pltpu
from jax.experimental.pallas import tpu_sc as plsc

VOCAB = 1000000
EMB = 32
BATCH = 16384
BAG = 50

NC, NS = 2, 16
NW = NC * NS
BAGS_PER_W = BATCH // NW
CHUNK = 64
ROWS = CHUNK * BAG
GRP = 128
NGRP = ROWS // GRP
NCHUNK = BAGS_PER_W // CHUNK
HALF = EMB // 2


def _body(idx_hbm, table_hbm, out_hbm, idx_v, rows_v, out_v, sem):
    wid = lax.axis_index("s") * NC + lax.axis_index("c")

    def chunk_body(c, carry):
        pltpu.sync_copy(idx_hbm.at[wid * NCHUNK + c], idx_v)
        copies = [
            pltpu.async_copy(
                table_hbm.at[idx_v.at[g]],
                rows_v.at[pl.ds(g * GRP, GRP)],
                sem,
            )
            for g in range(NGRP)
        ]
        for cp in copies:
            cp.wait()

        def bag_body(b, _):
            base = b * BAG

            def j_body(j, accs):
                a0, a1 = accs
                r = base + j
                a0 = a0 + rows_v[r, pl.ds(0, HALF)]
                a1 = a1 + rows_v[r, pl.ds(HALF, HALF)]
                return a0, a1

            zero = jnp.zeros((HALF,), jnp.float32)
            a0, a1 = lax.fori_loop(0, BAG, j_body, (zero, zero))
            out_v[b, pl.ds(0, HALF)] = a0
            out_v[b, pl.ds(HALF, HALF)] = a1
            return _

        lax.fori_loop(0, CHUNK, bag_body, 0)
        pltpu.sync_copy(out_v, out_hbm.at[pl.ds(wid * BAGS_PER_W + c * CHUNK, CHUNK)])
        return carry

    lax.fori_loop(0, NCHUNK, chunk_body, 0)


@jax.jit
def _run(idx_grouped, table):
    mesh = plsc.VectorSubcoreMesh(core_axis_name="c", subcore_axis_name="s")
    f = pl.kernel(
        _body,
        out_type=jax.ShapeDtypeStruct((BATCH, EMB), jnp.float32),
        mesh=mesh,
        scratch_types=[
            pltpu.VMEM((NGRP, GRP), jnp.int32),
            pltpu.VMEM((ROWS, EMB), jnp.float32),
            pltpu.VMEM((CHUNK, EMB), jnp.float32),
            pltpu.SemaphoreType.DMA,
        ],
    )
    return f(idx_grouped, table)


def kernel(indices, table):
    idx_grouped = indices.reshape(NW * NCHUNK, NGRP, GRP)
    return _run(idx_grouped, table)


# speedup vs baseline: 2.5533x; 2.5533x over previous
import functools

import jax
import jax.numpy as jnp
from jax import lax
from jax.experimental import pallas as pl
from jax.experimental.pallas import tpu as pltpu
from jax.experimental.pallas import tpu_sc as plsc

VOCAB = 1000000
EMB = 32
BATCH = 16384
BAG = 50

NC, NS = 2, 16
NW = NC * NS
BAGS_PER_W = BATCH // NW
CHUNK = 64
ROWS = CHUNK * BAG
GRP = 128
NGRP = ROWS // GRP
NCHUNK = BAGS_PER_W // CHUNK
HALF = EMB // 2


def _body(idx_hbm, table_hbm, out_hbm, idx_v, rows_v, out_v, sem):
    wid = lax.axis_index("s") * NC + lax.axis_index("c")

    def chunk_body(c, carry):
        pltpu.sync_copy(idx_hbm.at[wid * NCHUNK + c], idx_v)
        copies = [
            pltpu.async_copy(
                table_hbm.at[idx_v.at[g]],
                rows_v.at[pl.ds(g * GRP, GRP)],
                sem,
            )
            for g in range(NGRP)
        ]
        for cp in copies:
            cp.wait()

        def bag_body(b, _):
            base = b * BAG

            def j_body(j, accs):
                a0, a1 = accs
                r = base + j
                a0 = a0 + rows_v[r, pl.ds(0, HALF)]
                a1 = a1 + rows_v[r, pl.ds(HALF, HALF)]
                return a0, a1

            zero = jnp.zeros((HALF,), jnp.float32)
            a0, a1 = lax.fori_loop(0, BAG, j_body, (zero, zero))
            out_v[b, pl.ds(0, HALF)] = a0
            out_v[b, pl.ds(HALF, HALF)] = a1
            return _

        lax.fori_loop(0, CHUNK, bag_body, 0)
        pltpu.sync_copy(out_v, out_hbm.at[pl.ds(wid * BAGS_PER_W + c * CHUNK, CHUNK)])
        return carry

    lax.fori_loop(0, NCHUNK, chunk_body, 0)


@jax.jit
def _run(idx_grouped, table):
    mesh = plsc.VectorSubcoreMesh(core_axis_name="c", subcore_axis_name="s")
    f = pl.kernel(
        _body,
        out_type=jax.ShapeDtypeStruct((BATCH, EMB), jnp.float32),
        mesh=mesh,
        scratch_types=[
            pltpu.VMEM((NGRP, GRP), jnp.int32),
            pltpu.VMEM((ROWS, EMB), jnp.float32),
            pltpu.VMEM((CHUNK, EMB), jnp.float32),
            pltpu.SemaphoreType.DMA,
        ],
        compiler_params=pltpu.CompilerParams(use_tc_tiling_on_sc=False),
    )
    return f(idx_grouped, table)


def kernel(indices, table):
    idx_grouped = indices.reshape(NW * NCHUNK, NGRP, GRP)
    return _run(idx_grouped, table)


# speedup vs baseline: 2.8029x; 1.0978x over previous
import functools

import jax
import jax.numpy as jnp
from jax import lax
from jax.experimental import pallas as pl
from jax.experimental.pallas import tpu as pltpu
from jax.experimental.pallas import tpu_sc as plsc

VOCAB = 1000000
EMB = 32
BATCH = 16384
BAG = 50

NC, NS = 2, 16
NW = NC * NS
BAGS_PER_W = BATCH // NW
CHUNK = 64
ROWS = CHUNK * BAG
GRP = 128
NGRP = ROWS // GRP
NCHUNK = BAGS_PER_W // CHUNK
HALF = EMB // 2


def _body(idx_hbm, table_hbm, out_hbm, idx_v, rows_v, out_v, sem):
    wid = lax.axis_index("s") * NC + lax.axis_index("c")

    def chunk_body(c, carry):
        pltpu.sync_copy(idx_hbm.at[wid * NCHUNK + c], idx_v)
        copies = [
            pltpu.async_copy(
                table_hbm.at[idx_v.at[g]],
                rows_v.at[pl.ds(g * GRP, GRP)],
                sem,
            )
            for g in range(NGRP)
        ]
        for cp in copies:
            cp.wait()

        def bag_body(b, _):
            base = b * BAG

            def j_body(j, accs):
                a0, a1 = accs
                r = base + j
                a0 = a0 + rows_v[r, pl.ds(0, HALF)]
                a1 = a1 + rows_v[r, pl.ds(HALF, HALF)]
                return a0, a1

            zero = jnp.zeros((HALF,), jnp.float32)
            a0, a1 = lax.fori_loop(0, BAG, j_body, (zero, zero), unroll=10)
            out_v[b, pl.ds(0, HALF)] = a0
            out_v[b, pl.ds(HALF, HALF)] = a1
            return _

        lax.fori_loop(0, CHUNK, bag_body, 0)
        pltpu.sync_copy(out_v, out_hbm.at[pl.ds(wid * BAGS_PER_W + c * CHUNK, CHUNK)])
        return carry

    lax.fori_loop(0, NCHUNK, chunk_body, 0)


@jax.jit
def _run(idx_grouped, table):
    mesh = plsc.VectorSubcoreMesh(core_axis_name="c", subcore_axis_name="s")
    f = pl.kernel(
        _body,
        out_type=jax.ShapeDtypeStruct((BATCH, EMB), jnp.float32),
        mesh=mesh,
        scratch_types=[
            pltpu.VMEM((NGRP, GRP), jnp.int32),
            pltpu.VMEM((ROWS, EMB), jnp.float32),
            pltpu.VMEM((CHUNK, EMB), jnp.float32),
            pltpu.SemaphoreType.DMA,
        ],
        compiler_params=pltpu.CompilerParams(use_tc_tiling_on_sc=False),
    )
    return f(idx_grouped, table)


def kernel(indices, table):
    idx_grouped = indices.reshape(NW * NCHUNK, NGRP, GRP)
    table_lin = lax.optimization_barrier(table.reshape(VOCAB * EMB))
    return _run(idx_grouped, table_lin.reshape(VOCAB, EMB))
